# baseline (speedup 1.0000x reference)
import functools

import jax
import jax.numpy as jnp
from jax.experimental import pallas as pl
from jax.experimental.pallas import tpu as pltpu

_NC = 4
_TRANS_DN = (((1,), (1,)), ((), ()))


def _permute_cast_kernel(w_ref, o_ref):
    o_ref[...] = w_ref[...].astype(jnp.bfloat16)


def _gi_matmul_kernel(x_ref, w_ref, b_ref, o_ref):
    o_ref[...] = (
        jax.lax.dot_general(
            x_ref[...],
            w_ref[...],
            _TRANS_DN,
            preferred_element_type=jnp.float32,
        )
        + b_ref[...]
    ).astype(jnp.bfloat16)


def _recurrent_kernel(gi_ref, w_ref, b_ref, h0_ref, o_ref, h_ref, *, steps, H):
    @pl.when(pl.program_id(0) == 0)
    def _init():
        h_ref[...] = h0_ref[...]

    C = H // _NC

    def group(j, h):
        gi16 = gi_ref[pl.ds(j * 16, 16), :].astype(jnp.float32)
        for k in range(16):
            h16 = h.astype(jnp.bfloat16)
            gi = gi16[k : k + 1, :]
            h_parts = []
            for c in range(_NC):
                lo = 3 * C * c
                g = (
                    jax.lax.dot_general(
                        h16,
                        w_ref[lo : lo + 3 * C, :],
                        _TRANS_DN,
                        preferred_element_type=jnp.float32,
                    )
                    + b_ref[:, lo : lo + 3 * C]
                )
                gic = gi[:, lo : lo + 3 * C]
                r = jax.nn.sigmoid(gic[:, :C] + g[:, :C])
                z = jax.nn.sigmoid(gic[:, C : 2 * C] + g[:, C : 2 * C])
                n = jnp.tanh(gic[:, 2 * C :] + r * g[:, 2 * C :])
                h_parts.append((1.0 - z) * n + z * h[:, c * C : (c + 1) * C])
            h = jnp.concatenate(h_parts, axis=1)
            o_ref[pl.ds(j * 16 + k, 1), :] = h
        return h

    h_ref[...] = jax.lax.fori_loop(0, steps // 16, group, h_ref[...])


def _permute_gate_cols(w, H):
    C = H // _NC
    rows = w.shape[0]
    return (
        w.reshape(rows, 3, _NC, C)
        .transpose(0, 2, 1, 3)
        .reshape(rows, 3 * H)
    )


def kernel(input_, hidden, batch_sizes, W_ih, W_hh, b_ih, b_hh):
    del batch_sizes
    T, D = input_.shape
    H = hidden.shape[1]
    G = 3 * H
    C = H // _NC

    b_ih_2d = _permute_gate_cols(b_ih.reshape(1, G), H)
    b_hh_2d = _permute_gate_cols(b_hh.reshape(1, G), H)

    def _perm_tile(j):
        return (j % 3) * _NC + j // 3

    w_hh_p = pl.pallas_call(
        _permute_cast_kernel,
        grid=(3 * _NC,),
        in_specs=[pl.BlockSpec((C, H), lambda j: (_perm_tile(j), 0))],
        out_specs=pl.BlockSpec((C, H), lambda j: (j, 0)),
        out_shape=jax.ShapeDtypeStruct((G, H), jnp.bfloat16),
    )(W_hh)

    TM, TN = 256, 256
    gi = pl.pallas_call(
        _gi_matmul_kernel,
        grid=(G // TN, T // TM),
        in_specs=[
            pl.BlockSpec((TM, D), lambda j, i: (i, 0)),
            pl.BlockSpec((TN, D), lambda j, i: (_perm_tile(j), 0)),
            pl.BlockSpec((1, TN), lambda j, i: (0, j)),
        ],
        out_specs=pl.BlockSpec((TM, TN), lambda j, i: (i, j)),
        out_shape=jax.ShapeDtypeStruct((T, G), jnp.bfloat16),
    )(input_.astype(jnp.bfloat16), W_ih.astype(jnp.bfloat16), b_ih_2d)

    S = 64
    out = pl.pallas_call(
        functools.partial(_recurrent_kernel, steps=S, H=H),
        grid=(T // S,),
        in_specs=[
            pl.BlockSpec((S, G), lambda i: (i, 0)),
            pl.BlockSpec((G, H), lambda i: (0, 0)),
            pl.BlockSpec((1, G), lambda i: (0, 0)),
            pl.BlockSpec((1, H), lambda i: (0, 0)),
        ],
        out_specs=pl.BlockSpec((S, H), lambda i: (i, 0)),
        out_shape=jax.ShapeDtypeStruct((T, H), jnp.float32),
        scratch_shapes=[pltpu.VMEM((1, H), jnp.float32)],
    )(gi, w_hh_p, b_hh_2d, hidden)

    final_hidden = jax.lax.slice_in_dim(out, T - 1, T, axis=0)
    return (final_hidden, out)


# speedup vs baseline: 1.9413x; 1.9413x over previous
import functools

import jax
import jax.numpy as jnp
from jax.experimental import pallas as pl
from jax.experimental.pallas import tpu as pltpu

_NC = 4


def _transpose_cast_kernel(w_ref, o_ref):
    o_ref[...] = jnp.swapaxes(w_ref[...], 0, 1).astype(jnp.bfloat16)


def _gi_matmul_kernel(x_ref, w_ref, b_ref, o_ref):
    o_ref[...] = (
        jnp.dot(x_ref[...], w_ref[...], preferred_element_type=jnp.float32)
        + b_ref[...]
    ).astype(jnp.bfloat16)


def _recurrent_kernel(gi_ref, w_ref, b_ref, h0_ref, o_ref, h_ref, *, steps, H):
    @pl.when(pl.program_id(0) == 0)
    def _init():
        h_ref[...] = h0_ref[...]

    C = H // _NC

    def group(j, h):
        gi16 = gi_ref[pl.ds(j * 16, 16), :].astype(jnp.float32)
        for k in range(16):
            h16 = h.astype(jnp.bfloat16)
            gi = gi16[k : k + 1, :]
            h_parts = []
            for c in range(_NC):
                lo = 3 * C * c
                g = (
                    jnp.dot(
                        h16,
                        w_ref[:, lo : lo + 3 * C],
                        preferred_element_type=jnp.float32,
                    )
                    + b_ref[:, lo : lo + 3 * C]
                )
                gic = gi[:, lo : lo + 3 * C]
                r = jax.nn.sigmoid(gic[:, :C] + g[:, :C])
                z = jax.nn.sigmoid(gic[:, C : 2 * C] + g[:, C : 2 * C])
                n = jnp.tanh(gic[:, 2 * C :] + r * g[:, 2 * C :])
                h_parts.append((1.0 - z) * n + z * h[:, c * C : (c + 1) * C])
            h = jnp.concatenate(h_parts, axis=1)
            o_ref[pl.ds(j * 16 + k, 1), :] = h
        return h

    h_ref[...] = jax.lax.fori_loop(0, steps // 16, group, h_ref[...])


def _permute_gate_cols(w, H):
    C = H // _NC
    rows = w.shape[0]
    return (
        w.reshape(rows, 3, _NC, C)
        .transpose(0, 2, 1, 3)
        .reshape(rows, 3 * H)
    )


def _transpose_permute(w, H):
    C = H // _NC
    in_dim = w.shape[1]
    return pl.pallas_call(
        _transpose_cast_kernel,
        grid=(3 * _NC,),
        in_specs=[
            pl.BlockSpec((C, in_dim), lambda j: ((j % 3) * _NC + j // 3, 0))
        ],
        out_specs=pl.BlockSpec((in_dim, C), lambda j: (0, j)),
        out_shape=jax.ShapeDtypeStruct((in_dim, 3 * H), jnp.bfloat16),
    )(w)


def kernel(input_, hidden, batch_sizes, W_ih, W_hh, b_ih, b_hh):
    del batch_sizes
    T, D = input_.shape
    H = hidden.shape[1]
    G = 3 * H

    w_ih_t = _transpose_permute(W_ih, H)
    w_hh_t = _transpose_permute(W_hh, H)
    b_ih_2d = _permute_gate_cols(b_ih.reshape(1, G), H)
    b_hh_2d = _permute_gate_cols(b_hh.reshape(1, G), H)

    TM, TN = 256, 1024
    gi = pl.pallas_call(
        _gi_matmul_kernel,
        grid=(G // TN, T // TM),
        in_specs=[
            pl.BlockSpec((TM, D), lambda j, i: (i, 0)),
            pl.BlockSpec((D, TN), lambda j, i: (0, j)),
            pl.BlockSpec((1, TN), lambda j, i: (0, j)),
        ],
        out_specs=pl.BlockSpec((TM, TN), lambda j, i: (i, j)),
        out_shape=jax.ShapeDtypeStruct((T, G), jnp.bfloat16),
    )(input_.astype(jnp.bfloat16), w_ih_t, b_ih_2d)

    S = 64
    out = pl.pallas_call(
        functools.partial(_recurrent_kernel, steps=S, H=H),
        grid=(T // S,),
        in_specs=[
            pl.BlockSpec((S, G), lambda i: (i, 0)),
            pl.BlockSpec((H, G), lambda i: (0, 0)),
            pl.BlockSpec((1, G), lambda i: (0, 0)),
            pl.BlockSpec((1, H), lambda i: (0, 0)),
        ],
        out_specs=pl.BlockSpec((S, H), lambda i: (i, 0)),
        out_shape=jax.ShapeDtypeStruct((T, H), jnp.float32),
        scratch_shapes=[pltpu.VMEM((1, H), jnp.float32)],
    )(gi, w_hh_t, b_hh_2d, hidden)

    final_hidden = jax.lax.slice_in_dim(out, T - 1, T, axis=0)
    return (final_hidden, out)


# speedup vs baseline: 1.9462x; 1.0026x over previous
import functools

import jax
import jax.numpy as jnp
from jax.experimental import pallas as pl
from jax.experimental.pallas import tpu as pltpu

_NC = 4


def _transpose_cast_kernel(w_ref, o_ref):
    o_ref[...] = jnp.swapaxes(w_ref[...], 0, 1).astype(jnp.bfloat16)


def _gi_matmul_kernel(x_ref, w_ref, b_ref, o_ref):
    o_ref[...] = (
        jnp.dot(
            x_ref[...].astype(jnp.bfloat16),
            w_ref[...],
            preferred_element_type=jnp.float32,
        )
        + b_ref[...]
    ).astype(jnp.bfloat16)


def _recurrent_kernel(gi_ref, w_ref, b_ref, h0_ref, o_ref, h_ref, *, steps, H):
    @pl.when(pl.program_id(0) == 0)
    def _init():
        h_ref[...] = h0_ref[...]

    C = H // _NC

    def group(j, h):
        gi16 = gi_ref[pl.ds(j * 16, 16), :].astype(jnp.float32)
        for k in range(16):
            h16 = h.astype(jnp.bfloat16)
            gi = gi16[k : k + 1, :]
            h_parts = []
            for c in range(_NC):
                lo = 3 * C * c
                g = (
                    jnp.dot(
                        h16,
                        w_ref[:, lo : lo + 3 * C],
                        preferred_element_type=jnp.float32,
                    )
                    + b_ref[:, lo : lo + 3 * C]
                )
                gic = gi[:, lo : lo + 3 * C]
                r = jax.nn.sigmoid(gic[:, :C] + g[:, :C])
                z = jax.nn.sigmoid(gic[:, C : 2 * C] + g[:, C : 2 * C])
                n = jnp.tanh(gic[:, 2 * C :] + r * g[:, 2 * C :])
                h_parts.append((1.0 - z) * n + z * h[:, c * C : (c + 1) * C])
            h = jnp.concatenate(h_parts, axis=1)
            o_ref[pl.ds(j * 16 + k, 1), :] = h
        return h

    h_ref[...] = jax.lax.fori_loop(0, steps // 16, group, h_ref[...])


def _permute_gate_cols(w, H):
    C = H // _NC
    rows = w.shape[0]
    return (
        w.reshape(rows, 3, _NC, C)
        .transpose(0, 2, 1, 3)
        .reshape(rows, 3 * H)
    )


def _transpose_permute(w, H):
    C = H // _NC
    in_dim = w.shape[1]
    return pl.pallas_call(
        _transpose_cast_kernel,
        grid=(3 * _NC,),
        in_specs=[
            pl.BlockSpec((C, in_dim), lambda j: ((j % 3) * _NC + j // 3, 0))
        ],
        out_specs=pl.BlockSpec((in_dim, C), lambda j: (0, j)),
        out_shape=jax.ShapeDtypeStruct((in_dim, 3 * H), jnp.bfloat16),
    )(w)


def kernel(input_, hidden, batch_sizes, W_ih, W_hh, b_ih, b_hh):
    del batch_sizes
    T, D = input_.shape
    H = hidden.shape[1]
    G = 3 * H

    w_ih_t = _transpose_permute(W_ih, H)
    w_hh_t = _transpose_permute(W_hh, H)
    b_ih_2d = _permute_gate_cols(b_ih.reshape(1, G), H)
    b_hh_2d = _permute_gate_cols(b_hh.reshape(1, G), H)

    TM, TN = 256, 1024
    gi = pl.pallas_call(
        _gi_matmul_kernel,
        grid=(G // TN, T // TM),
        in_specs=[
            pl.BlockSpec((TM, D), lambda j, i: (i, 0)),
            pl.BlockSpec((D, TN), lambda j, i: (0, j)),
            pl.BlockSpec((1, TN), lambda j, i: (0, j)),
        ],
        out_specs=pl.BlockSpec((TM, TN), lambda j, i: (i, j)),
        out_shape=jax.ShapeDtypeStruct((T, G), jnp.bfloat16),
    )(input_, w_ih_t, b_ih_2d)

    S = 64
    out = pl.pallas_call(
        functools.partial(_recurrent_kernel, steps=S, H=H),
        grid=(T // S,),
        in_specs=[
            pl.BlockSpec((S, G), lambda i: (i, 0)),
            pl.BlockSpec((H, G), lambda i: (0, 0)),
            pl.BlockSpec((1, G), lambda i: (0, 0)),
            pl.BlockSpec((1, H), lambda i: (0, 0)),
        ],
        out_specs=pl.BlockSpec((S, H), lambda i: (i, 0)),
        out_shape=jax.ShapeDtypeStruct((T, H), jnp.float32),
        scratch_shapes=[pltpu.VMEM((1, H), jnp.float32)],
    )(gi, w_hh_t, b_hh_2d, hidden)

    final_hidden = jax.lax.slice_in_dim(out, T - 1, T, axis=0)
    return (final_hidden, out)


# speedup vs baseline: 1.9543x; 1.0041x over previous
import functools

import jax
import jax.numpy as jnp
from jax.experimental import pallas as pl
from jax.experimental.pallas import tpu as pltpu

_NC = 4


def _transpose_cast_kernel(w_ref, o_ref):
    o_ref[...] = jnp.swapaxes(w_ref[...], 0, 1).astype(jnp.bfloat16)


def _gi_matmul_kernel(x_ref, w_ref, b_ref, o_ref):
    o_ref[...] = (
        jnp.dot(
            x_ref[...].astype(jnp.bfloat16),
            w_ref[...],
            preferred_element_type=jnp.float32,
        )
        + b_ref[...]
    ).astype(jnp.bfloat16)


def _recurrent_kernel(gi_ref, w_ref, b_ref, h0_ref, o_ref, h_ref, *, steps, H):
    @pl.when(pl.program_id(0) == 0)
    def _init():
        h_ref[...] = h0_ref[...]

    C = H // _NC

    def group(j, h):
        gi16 = gi_ref[pl.ds(j * 32, 32), :].astype(jnp.float32)
        for k in range(32):
            h16 = h.astype(jnp.bfloat16)
            gi = gi16[k : k + 1, :]
            h_parts = []
            for c in range(_NC):
                lo = 3 * C * c
                g = (
                    jnp.dot(
                        h16,
                        w_ref[:, lo : lo + 3 * C],
                        preferred_element_type=jnp.float32,
                    )
                    + b_ref[:, lo : lo + 3 * C]
                )
                gic = gi[:, lo : lo + 3 * C]
                r = jax.nn.sigmoid(gic[:, :C] + g[:, :C])
                z = jax.nn.sigmoid(gic[:, C : 2 * C] + g[:, C : 2 * C])
                n = jnp.tanh(gic[:, 2 * C :] + r * g[:, 2 * C :])
                h_parts.append((1.0 - z) * n + z * h[:, c * C : (c + 1) * C])
            h = jnp.concatenate(h_parts, axis=1)
            o_ref[pl.ds(j * 32 + k, 1), :] = h
        return h

    h_ref[...] = jax.lax.fori_loop(0, steps // 32, group, h_ref[...])


def _permute_gate_cols(w, H):
    C = H // _NC
    rows = w.shape[0]
    return (
        w.reshape(rows, 3, _NC, C)
        .transpose(0, 2, 1, 3)
        .reshape(rows, 3 * H)
    )


def _transpose_permute(w, H):
    C = H // _NC
    in_dim = w.shape[1]
    return pl.pallas_call(
        _transpose_cast_kernel,
        grid=(3 * _NC,),
        in_specs=[
            pl.BlockSpec((C, in_dim), lambda j: ((j % 3) * _NC + j // 3, 0))
        ],
        out_specs=pl.BlockSpec((in_dim, C), lambda j: (0, j)),
        out_shape=jax.ShapeDtypeStruct((in_dim, 3 * H), jnp.bfloat16),
    )(w)


def kernel(input_, hidden, batch_sizes, W_ih, W_hh, b_ih, b_hh):
    del batch_sizes
    T, D = input_.shape
    H = hidden.shape[1]
    G = 3 * H

    w_ih_t = _transpose_permute(W_ih, H)
    w_hh_t = _transpose_permute(W_hh, H)
    b_ih_2d = _permute_gate_cols(b_ih.reshape(1, G), H)
    b_hh_2d = _permute_gate_cols(b_hh.reshape(1, G), H)

    TM, TN = 256, 1024
    gi = pl.pallas_call(
        _gi_matmul_kernel,
        grid=(G // TN, T // TM),
        in_specs=[
            pl.BlockSpec((TM, D), lambda j, i: (i, 0)),
            pl.BlockSpec((D, TN), lambda j, i: (0, j)),
            pl.BlockSpec((1, TN), lambda j, i: (0, j)),
        ],
        out_specs=pl.BlockSpec((TM, TN), lambda j, i: (i, j)),
        out_shape=jax.ShapeDtypeStruct((T, G), jnp.bfloat16),
    )(input_, w_ih_t, b_ih_2d)

    S = 64
    out = pl.pallas_call(
        functools.partial(_recurrent_kernel, steps=S, H=H),
        grid=(T // S,),
        in_specs=[
            pl.BlockSpec((S, G), lambda i: (i, 0)),
            pl.BlockSpec((H, G), lambda i: (0, 0)),
            pl.BlockSpec((1, G), lambda i: (0, 0)),
            pl.BlockSpec((1, H), lambda i: (0, 0)),
        ],
        out_specs=pl.BlockSpec((S, H), lambda i: (i, 0)),
        out_shape=jax.ShapeDtypeStruct((T, H), jnp.float32),
        scratch_shapes=[pltpu.VMEM((1, H), jnp.float32)],
    )(gi, w_hh_t, b_hh_2d, hidden)

    final_hidden = jax.lax.slice_in_dim(out, T - 1, T, axis=0)
    return (final_hidden, out)
